# Initial kernel scaffold; baseline (speedup 1.0000x reference)
import jax
import jax.numpy as jnp
from jax.experimental import pallas as pl


def kernel(word_sequences, embedding_table):
    raise NotImplementedError("write your pallas kernel here")


# speedup vs baseline: 4.3377x; 4.3377x over previous
import functools

import jax
import jax.numpy as jnp
from jax import lax
from jax.experimental import pallas as pl
from jax.experimental.pallas import tpu as pltpu
from jax.experimental.pallas import tpu_sc as plsc

VOCAB = 1000
D = 64
BATCH = 4096
SEQ = 200
VPAD = 1024

NC = 2
NS = 16
NW = NC * NS
IDX_PER_W = SEQ * 128


def _body(tabf_hbm, idx_hbm, out_hbm, tab_v, idx_v, buf0, buf1, sem0, sem1):
    wid = lax.axis_index("s") * NC + lax.axis_index("c")
    pltpu.sync_copy(tabf_hbm, tab_v)
    pltpu.sync_copy(idx_hbm.at[pl.ds(wid * IDX_PER_W, IDX_PER_W)], idx_v)

    bufs = (buf0, buf1)
    sems = (sem0, sem1)
    col0 = wid * 128

    def fill(s, buf):
        for j in range(8):
            idxv = idx_v[pl.ds(s * 128 + j * 16, 16)]
            for d in range(D):
                col = plsc.load_gather(tab_v, [idxv + d * VPAD])
                buf[d, pl.ds(j * 16, 16)] = col

    def start_out(s, b):
        pltpu.async_copy(bufs[b], out_hbm.at[s, :, pl.ds(col0, 128)], sems[b])

    def wait_out(s, b):
        pltpu.make_async_copy(
            bufs[b], out_hbm.at[s, :, pl.ds(col0, 128)], sems[b]
        ).wait()

    fill(0, bufs[0])
    start_out(0, 0)
    fill(1, bufs[1])
    start_out(1, 1)

    def step(i, _):
        for b in range(2):
            s = 2 + 2 * i + b
            wait_out(s - 2, b)
            fill(s, bufs[b])
            start_out(s, b)
        return 0

    lax.fori_loop(0, (SEQ - 2) // 2, step, 0, unroll=False)
    wait_out(SEQ - 2, 0)
    wait_out(SEQ - 1, 1)


def _lookup(tabf, idxf):
    mesh = plsc.VectorSubcoreMesh(core_axis_name="c", subcore_axis_name="s")
    f = functools.partial(
        pl.kernel,
        mesh=mesh,
        out_type=jax.ShapeDtypeStruct((SEQ, D, BATCH), jnp.float32),
        scratch_types=[
            pltpu.VMEM((D * VPAD,), jnp.float32),
            pltpu.VMEM((IDX_PER_W,), jnp.int32),
            pltpu.VMEM((D, 128), jnp.float32),
            pltpu.VMEM((D, 128), jnp.float32),
            pltpu.SemaphoreType.DMA,
            pltpu.SemaphoreType.DMA,
        ],
        compiler_params=pltpu.CompilerParams(
            use_tc_tiling_on_sc=True, needs_layout_passes=False
        ),
    )(_body)
    return f(tabf, idxf)


@jax.jit
def kernel(word_sequences, embedding_table):
    tabf = (
        jnp.zeros((D, VPAD), jnp.float32)
        .at[:, :VOCAB]
        .set(embedding_table.astype(jnp.float32).T)
        .reshape(-1)
    )
    idxf = (
        word_sequences.astype(jnp.int32)
        .reshape(NW, 128, SEQ)
        .transpose(0, 2, 1)
        .reshape(-1)
    )
    out = _lookup(tabf, idxf)
    return out.transpose(2, 0, 1)


# speedup vs baseline: 14.0625x; 3.2419x over previous
import functools

import jax
import jax.numpy as jnp
from jax import lax
from jax.experimental import pallas as pl
from jax.experimental.pallas import tpu as pltpu
from jax.experimental.pallas import tpu_sc as plsc

VOCAB = 1000
D = 64
BATCH = 4096
SEQ = 200
VPAD = 1024

NC = 2
NS = 16
NW = NC * NS
IDX_PER_W = SEQ * 128


def _body(tabf_hbm, idx_hbm, out_hbm, tab_v, idx_v, buf0, buf1, sem0, sem1):
    wid = lax.axis_index("s") * NC + lax.axis_index("c")
    pltpu.sync_copy(tabf_hbm, tab_v)
    pltpu.sync_copy(idx_hbm.at[pl.ds(wid * IDX_PER_W, IDX_PER_W)], idx_v)

    bufs = (buf0, buf1)
    sems = (sem0, sem1)
    col0 = wid * 128

    def fill(s, buf):
        for j in range(8):
            idxv = idx_v[pl.ds(s * 128 + j * 16, 16)]

            @plsc.parallel_loop(0, D, unroll=8)
            def _(d):
                col = plsc.load_gather(tab_v, [idxv + d * VPAD])
                buf[d, pl.ds(j * 16, 16)] = col

    def start_out(s, b):
        pltpu.async_copy(bufs[b], out_hbm.at[s, :, pl.ds(col0, 128)], sems[b])

    def wait_out(s, b):
        pltpu.make_async_copy(
            bufs[b], out_hbm.at[s, :, pl.ds(col0, 128)], sems[b]
        ).wait()

    fill(0, bufs[0])
    start_out(0, 0)
    fill(1, bufs[1])
    start_out(1, 1)

    def step(i, _):
        for b in range(2):
            s = 2 + 2 * i + b
            wait_out(s - 2, b)
            fill(s, bufs[b])
            start_out(s, b)
        return 0

    lax.fori_loop(0, (SEQ - 2) // 2, step, 0, unroll=False)
    wait_out(SEQ - 2, 0)
    wait_out(SEQ - 1, 1)


def _lookup(tabf, idxf):
    mesh = plsc.VectorSubcoreMesh(core_axis_name="c", subcore_axis_name="s")
    f = functools.partial(
        pl.kernel,
        mesh=mesh,
        out_type=jax.ShapeDtypeStruct((SEQ, D, BATCH), jnp.float32),
        scratch_types=[
            pltpu.VMEM((D * VPAD,), jnp.float32),
            pltpu.VMEM((IDX_PER_W,), jnp.int32),
            pltpu.VMEM((D, 128), jnp.float32),
            pltpu.VMEM((D, 128), jnp.float32),
            pltpu.SemaphoreType.DMA,
            pltpu.SemaphoreType.DMA,
        ],
        compiler_params=pltpu.CompilerParams(
            use_tc_tiling_on_sc=True, needs_layout_passes=False
        ),
    )(_body)
    return f(tabf, idxf)


@jax.jit
def kernel(word_sequences, embedding_table):
    tabf = (
        jnp.zeros((D, VPAD), jnp.float32)
        .at[:, :VOCAB]
        .set(embedding_table.astype(jnp.float32).T)
        .reshape(-1)
    )
    idxf = (
        word_sequences.astype(jnp.int32)
        .reshape(NW, 128, SEQ)
        .transpose(0, 2, 1)
        .reshape(-1)
    )
    out = _lookup(tabf, idxf)
    return out.transpose(2, 0, 1)
